# Initial kernel scaffold; baseline (speedup 1.0000x reference)
import jax
import jax.numpy as jnp
from jax.experimental import pallas as pl


def kernel(x, pos, edge_index, batch, W_dm, b_dm, W_mw1, b_mw1, W_mw2, b_mw2, W_msg, b_msg, W_upd, b_upd, W_o1, b_o1, W_o2, b_o2, W_o3, b_o3):
    raise NotImplementedError("write your pallas kernel here")


# speedup vs baseline: 4.5956x; 4.5956x over previous
import functools

import jax
import jax.numpy as jnp
from jax import lax
from jax.experimental import pallas as pl
from jax.experimental.pallas import tpu as pltpu
from jax.experimental.pallas import tpu_sc as plsc

N = 10000
E = 320000
D = 128
H = 128

NC = 2
NS = 16
NW = NC * NS
EP = E // NW
CH = 80
NCHUNK = EP // CH
NP = 10240
RPT = NP // NS
ZR = 8

_F32 = jnp.float32
_I32 = jnp.int32


def _xw_body(x_ref, w_ref, b_ref, o_ref):
    o_ref[...] = lax.dot_general(
        x_ref[...], w_ref[...], (((1,), (0,)), ((), ())),
        preferred_element_type=_F32) + b_ref[...]


def _stage_a(x, w_x, b_msg):
    blk = 1000
    return pl.pallas_call(
        _xw_body,
        grid=(N // blk,),
        in_specs=[
            pl.BlockSpec((blk, D), lambda i: (i, 0)),
            pl.BlockSpec((D, H), lambda i: (0, 0)),
            pl.BlockSpec((1, H), lambda i: (0, 0)),
        ],
        out_specs=pl.BlockSpec((blk, H), lambda i: (i, 0)),
        out_shape=jax.ShapeDtypeStruct((N, H), _F32),
    )(x, w_x, b_msg)


def _iota16():
    return lax.broadcasted_iota(_I32, (16,), 0)


def _rne_bf16(v):
    u = plsc.bitcast(v, _I32)
    lsb = lax.shift_right_logical(u, 16) & 1
    r = (u + 0x7FFF + lsb) & jnp.int32(-65536)
    return plsc.bitcast(r, _F32)


def _make_edge_kernel():
    mesh = plsc.VectorSubcoreMesh(core_axis_name="c", subcore_axis_name="s")

    @functools.partial(
        pl.kernel,
        mesh=mesh,
        compiler_params=pltpu.CompilerParams(needs_layout_passes=False),
        out_type=jax.ShapeDtypeStruct((NC, NP, H), _F32),
        scratch_types=[
            pltpu.VMEM((CH,), _I32),
            pltpu.VMEM((CH,), _I32),
            pltpu.VMEM((CH, H), _F32),
            pltpu.VMEM((CH,), _F32),
            pltpu.VMEM((CH,), _F32),
            pltpu.VMEM((CH,), _F32),
            pltpu.VMEM((CH,), _F32),
            pltpu.VMEM((CH,), _F32),
            pltpu.VMEM((CH,), _F32),
            pltpu.VMEM((4, H), _F32),
            pltpu.VMEM((8, 16), _F32),
            pltpu.VMEM((8, 16), _F32),
            pltpu.VMEM((8, 16, 16), _F32),
            pltpu.VMEM((16, 16), _F32),
            pltpu.VMEM((16, 16), _F32),
            pltpu.VMEM((16,), _F32),
            pltpu.VMEM((ZR, H), _F32),
            pltpu.VMEM_SHARED((NP, H), _F32),
            pltpu.SemaphoreType.DMA,
            pltpu.SemaphoreType.DMA,
            pltpu.SemaphoreType.DMA,
        ],
    )
    def kfn(xw_hbm, px_hbm, py_hbm, pz_hbm, src_hbm, dst_hbm, we_hbm,
            wdm_hbm, bdm_hbm, w1_hbm, b1_hbm, w2_hbm, b2_hbm,
            out_hbm,
            si_v, di_v, rows_v, psx_v, psy_v, psz_v,
            pdx_v, pdy_v, pdz_v, we_v,
            wdm_v, bdm_v, w1_v, b1_v, w2_v, b2_v, zb_v, agg_sh,
            sem0, sem1, sem2):
        c = lax.axis_index("c")
        s = lax.axis_index("s")
        wid = s * NC + c

        pltpu.sync_copy(we_hbm, we_v)
        pltpu.sync_copy(wdm_hbm, wdm_v)
        pltpu.sync_copy(bdm_hbm, bdm_v)
        pltpu.sync_copy(w1_hbm, w1_v)
        pltpu.sync_copy(b1_hbm, b1_v)
        pltpu.sync_copy(w2_hbm, w2_v)
        pltpu.sync_copy(b2_hbm, b2_v)

        zvec = jnp.zeros((16,), _F32)

        def _zrow(i, carry):
            for q in range(H // 16):
                zb_v[i, pl.ds(q * 16, 16)] = zvec
            return carry

        lax.fori_loop(0, ZR, _zrow, 0)

        row0 = s * RPT
        for j in range(RPT // ZR):
            pltpu.sync_copy(zb_v, agg_sh.at[pl.ds(row0 + j * ZR, ZR)])
        plsc.subcore_barrier()

        it16 = _iota16()
        wel = [[we_v[p, pl.ds(r * 16, 16)] for p in range(4)] for r in range(8)]
        b2vec = b2_v[...]

        def _chunk(i, carry):
            base = wid * EP + i * CH
            pltpu.sync_copy(src_hbm.at[pl.ds(base, CH)], si_v)
            pltpu.sync_copy(dst_hbm.at[pl.ds(base, CH)], di_v)
            cg = pltpu.async_copy(xw_hbm.at[si_v], rows_v, sem0)
            c1 = pltpu.async_copy(px_hbm.at[si_v], psx_v, sem1)
            c2 = pltpu.async_copy(py_hbm.at[si_v], psy_v, sem1)
            c3 = pltpu.async_copy(pz_hbm.at[si_v], psz_v, sem1)
            c4 = pltpu.async_copy(px_hbm.at[di_v], pdx_v, sem2)
            c5 = pltpu.async_copy(py_hbm.at[di_v], pdy_v, sem2)
            c6 = pltpu.async_copy(pz_hbm.at[di_v], pdz_v, sem2)
            c1.wait()
            c2.wait()
            c3.wait()
            c4.wait()
            c5.wait()
            c6.wait()
            cg.wait()

            def _grp(g, carry2):
                off = pl.ds(g * 16, 16)
                r0 = psx_v[off] - pdx_v[off]
                r1 = psy_v[off] - pdy_v[off]
                r2 = psz_v[off] - pdz_v[off]
                d2 = r0 * r0 + r1 * r1 + r2 * r2
                d2c = jnp.maximum(d2, 1e-30)
                yi = jnp.int32(0x5F3759DF) - lax.shift_right_logical(
                    plsc.bitcast(d2c, _I32), 1)
                y = plsc.bitcast(yi, _F32)
                for _ in range(3):
                    y = y * (1.5 - 0.5 * d2c * y * y)
                dd = d2 * y

                accs = [b1_v[k, :] for k in range(16)]
                for j in range(8):
                    hj = _rne_bf16(dd * wdm_v[j, :] + bdm_v[j, :])
                    for k in range(16):
                        accs[k] = accs[k] + hj * w1_v[j, k, :]
                logits = b2vec
                for k in range(16):
                    hk = _rne_bf16(jnp.maximum(accs[k], 0.0))
                    logits = logits + hk * w2_v[k, :]
                attn = 1.0 / (1.0 + jnp.exp(-logits))

                r0 = _rne_bf16(r0)
                r1 = _rne_bf16(r1)
                r2 = _rne_bf16(r2)
                ddb = _rne_bf16(dd)

                for l in range(16):
                    e = g * 16 + l
                    a0 = r0[l]
                    a1 = r1[l]
                    a2 = r2[l]
                    a3 = ddb[l]
                    at = attn[l]
                    for r in range(8):
                        off2 = pl.ds(r * 16, 16)
                        v = rows_v[e, off2]
                        v = (v + a0 * wel[r][0] + a1 * wel[r][1]
                             + a2 * wel[r][2] + a3 * wel[r][3])
                        rows_v[e, off2] = _rne_bf16(jnp.maximum(v, 0.0) * at)
                return carry2

            lax.fori_loop(0, CH // 16, _grp, 0)

            pltpu.sync_copy(rows_v, agg_sh.at[di_v], add=True)
            return carry

        lax.fori_loop(0, NCHUNK, _chunk, 0)

        plsc.subcore_barrier()

        for j in range(RPT // ZR):
            r0_ = row0 + j * ZR
            pltpu.sync_copy(agg_sh.at[pl.ds(r0_, ZR)], out_hbm.at[c, pl.ds(r0_, ZR)])

    return kfn


def _upd_body(x_ref, agg_ref, b3_ref, wux_ref, wua_ref, bu_ref,
              wo1_ref, bo1_ref, wo2_ref, bo2_ref, wo3_ref, bo3_ref,
              o_ref, acc_ref, cnt_ref):
    i = pl.program_id(0)
    nb = pl.num_programs(0)

    @pl.when(i == 0)
    def _init():
        acc_ref[...] = jnp.zeros_like(acc_ref)
        cnt_ref[...] = jnp.zeros_like(cnt_ref)

    agg = agg_ref[0] + agg_ref[1]
    hp = lax.Precision.HIGHEST
    emb = (lax.dot_general(x_ref[...], wux_ref[...], (((1,), (0,)), ((), ())),
                           preferred_element_type=_F32)
           + lax.dot_general(agg, wua_ref[...], (((1,), (0,)), ((), ())),
                             preferred_element_type=_F32)
           + bu_ref[...])
    emb = jnp.maximum(emb, 0.0)

    bvec = b3_ref[0]
    gio = lax.broadcasted_iota(_I32, (64, bvec.shape[1]), 0)
    oh = (bvec == gio).astype(_F32)
    acc_ref[...] += lax.dot_general(oh, emb, (((1,), (0,)), ((), ())),
                                    precision=hp, preferred_element_type=_F32)
    cnt_ref[...] += jnp.sum(oh, axis=1, keepdims=True)

    @pl.when(i == nb - 1)
    def _fin():
        pool = acc_ref[...] / jnp.maximum(cnt_ref[...], 1.0)
        h1 = jnp.maximum(
            lax.dot_general(pool, wo1_ref[...], (((1,), (0,)), ((), ())),
                            preferred_element_type=_F32)
            + bo1_ref[...], 0.0)
        h2 = jnp.maximum(
            lax.dot_general(h1, wo2_ref[...], (((1,), (0,)), ((), ())),
                            preferred_element_type=_F32)
            + bo2_ref[...], 0.0)
        o_ref[...] = (lax.dot_general(h2, wo3_ref[...], (((1,), (0,)), ((), ())),
                                      preferred_element_type=_F32)
                      + bo3_ref[...])


def _stage_c(x, agg2, batch3, wux, wua, bu, wo1, bo1, wo2, bo2, wo3p, bo3p):
    blk = 1000
    nb = N // blk
    return pl.pallas_call(
        _upd_body,
        grid=(nb,),
        in_specs=[
            pl.BlockSpec((blk, D), lambda i: (i, 0)),
            pl.BlockSpec((NC, blk, H), lambda i: (0, i, 0)),
            pl.BlockSpec((1, 1, blk), lambda i: (i, 0, 0)),
            pl.BlockSpec((D, H), lambda i: (0, 0)),
            pl.BlockSpec((H, H), lambda i: (0, 0)),
            pl.BlockSpec((1, H), lambda i: (0, 0)),
            pl.BlockSpec((H, 2 * H), lambda i: (0, 0)),
            pl.BlockSpec((1, 2 * H), lambda i: (0, 0)),
            pl.BlockSpec((2 * H, H), lambda i: (0, 0)),
            pl.BlockSpec((1, H), lambda i: (0, 0)),
            pl.BlockSpec((H, H), lambda i: (0, 0)),
            pl.BlockSpec((1, H), lambda i: (0, 0)),
        ],
        out_specs=pl.BlockSpec((64, H), lambda i: (0, 0)),
        out_shape=jax.ShapeDtypeStruct((64, H), _F32),
        scratch_shapes=[
            pltpu.VMEM((64, H), _F32),
            pltpu.VMEM((64, 1), _F32),
        ],
    )(x, agg2, batch3, wux, wua, bu, wo1, bo1, wo2, bo2, wo3p, bo3p)


def kernel(x, pos, edge_index, batch,
           W_dm, b_dm, W_mw1, b_mw1, W_mw2, b_mw2,
           W_msg, b_msg, W_upd, b_upd,
           W_o1, b_o1, W_o2, b_o2, W_o3, b_o3):
    src = edge_index[0].astype(_I32)
    dst = edge_index[1].astype(_I32)
    px = pos[:, 0]
    py = pos[:, 1]
    pz = pos[:, 2]

    w_x = W_msg[:D]
    w_e = W_msg[D:].astype(jnp.bfloat16).astype(_F32)
    w1b = W_mw1.astype(jnp.bfloat16).astype(_F32)
    w2b = W_mw2[:, 0].astype(jnp.bfloat16).astype(_F32)
    wdm_t = jnp.broadcast_to(W_dm[0][:, None], (8, 16))
    bdm_t = jnp.broadcast_to(b_dm[:, None], (8, 16))
    w1_t = jnp.broadcast_to(w1b[:, :, None], (8, 16, 16))
    b1_t = jnp.broadcast_to(b_mw1[:, None], (16, 16))
    w2_t = jnp.broadcast_to(w2b[:, None], (16, 16))
    b2v = jnp.full((16,), b_mw2[0], _F32)

    xw = _stage_a(x, w_x, b_msg[None, :])

    edge_k = _make_edge_kernel()
    agg2 = edge_k(xw, px, py, pz, src, dst, w_e,
                  wdm_t, bdm_t, w1_t, b1_t, w2_t, b2v)

    batch3 = batch.astype(_I32).reshape(N // 1000, 1, 1000)
    wux = W_upd[:D]
    wua = W_upd[D:]
    wo3p = jnp.concatenate([W_o3, jnp.zeros((H, H - 1), _F32)], axis=1)
    bo3p = jnp.concatenate([b_o3, jnp.zeros((H - 1,), _F32)])[None, :]

    outp = _stage_c(x, agg2, batch3, wux, wua, b_upd[None, :],
                    W_o1, b_o1[None, :], W_o2, b_o2[None, :], wo3p, bo3p)
    return outp[:, :1]
